# Initial kernel scaffold; baseline (speedup 1.0000x reference)
import jax
import jax.numpy as jnp
from jax.experimental import pallas as pl


def kernel(d_mat, embeddings_table):
    raise NotImplementedError("write your pallas kernel here")


# speedup vs baseline: 4.7528x; 4.7528x over previous
import functools

import jax
import jax.numpy as jnp
from jax import lax
from jax.experimental import pallas as pl
from jax.experimental.pallas import tpu as pltpu
from jax.experimental.pallas import tpu_sc as plsc

MIN_POS_K = 0.1
MAX_POS_K = 1000.0
N_POS_EMB_K = 512
N_HEADS_K = 12

NC = 2
NS = 16
NW = NC * NS
LANES = 16

CHUNK = 2048
TAB_PAD = 6272


def _idx_body(d_ref, idx_ref):
    mn_log = jnp.log10(jnp.float32(MIN_POS_K))
    mx_log = jnp.log10(jnp.float32(MAX_POS_K))
    t = jnp.clip(d_ref[...], MIN_POS_K, MAX_POS_K)
    t = jnp.log10(t)
    t = (t - mn_log) / (mx_log - mn_log)
    t = N_POS_EMB_K * t
    idx_ref[...] = t.astype(jnp.int32) * N_HEADS_K


def _compute_idx(d2):
    rows, cols = d2.shape
    br = 256
    return pl.pallas_call(
        _idx_body,
        grid=(rows // br,),
        in_specs=[pl.BlockSpec((br, cols), lambda i: (i, 0))],
        out_specs=pl.BlockSpec((br, cols), lambda i: (i, 0)),
        out_shape=jax.ShapeDtypeStruct((rows, cols), jnp.int32),
    )(d2)


def _sc_gather(idx1, table_flat, total):
    b_per_w = total // NW
    nchunks = b_per_w // CHUNK
    mesh = plsc.VectorSubcoreMesh(
        core_axis_name="c", subcore_axis_name="s", num_cores=NC, num_subcores=NS
    )

    @functools.partial(
        pl.kernel,
        out_type=jax.ShapeDtypeStruct((total * N_HEADS_K,), jnp.float32),
        mesh=mesh,
        compiler_params=pltpu.CompilerParams(needs_layout_passes=False),
        scratch_types=[
            pltpu.VMEM((TAB_PAD,), jnp.float32),
            pltpu.VMEM((CHUNK,), jnp.int32),
            pltpu.VMEM((CHUNK * N_HEADS_K,), jnp.float32),
        ],
    )
    def run(idx_hbm, table_hbm, out_hbm, tab_v, idx_v, rows_v):
        wid = lax.axis_index("s") * NC + lax.axis_index("c")
        base_w = wid * b_per_w
        pltpu.sync_copy(table_hbm, tab_v)
        lane12 = lax.iota(jnp.int32, LANES) * N_HEADS_K

        @pl.loop(0, nchunks)
        def _(i):
            base = pl.multiple_of(base_w + i * CHUNK, CHUNK)
            pltpu.sync_copy(idx_hbm.at[pl.ds(base, CHUNK)], idx_v)

            @pl.loop(0, CHUNK // LANES)
            def _(p):
                f = idx_v[pl.ds(p * LANES, LANES)]
                dst = lane12 + p * (LANES * N_HEADS_K)
                for h in range(N_HEADS_K):
                    vals = plsc.load_gather(tab_v, [f + h])
                    plsc.store_scatter(rows_v, [dst + h], vals)

            pltpu.sync_copy(rows_v, out_hbm.at[pl.ds(base * N_HEADS_K, CHUNK * N_HEADS_K)])

    return run(idx1, table_flat)


def kernel(d_mat, embeddings_table):
    b, rows, cols = d_mat.shape
    total = b * rows * cols
    idx = _compute_idx(d_mat.reshape(b * rows, cols))
    tab_flat = jnp.zeros((TAB_PAD,), jnp.float32).at[: (N_POS_EMB_K + 1) * N_HEADS_K].set(
        embeddings_table.reshape(-1)
    )
    out = _sc_gather(idx.reshape(total), tab_flat, total)
    return out.reshape(b, rows, cols, N_HEADS_K)


# speedup vs baseline: 7.1322x; 1.5006x over previous
import functools

import jax
import jax.numpy as jnp
from jax import lax
from jax.experimental import pallas as pl
from jax.experimental.pallas import tpu as pltpu
from jax.experimental.pallas import tpu_sc as plsc

MIN_POS_K = 0.1
MAX_POS_K = 1000.0
N_POS_EMB_K = 512
N_HEADS_K = 12

NC = 2
NS = 16
NW = NC * NS
LANES = 16

CHUNK = 2048
SUB = 128
TAB_PAD = 6272


def _idx_body(d_ref, idx_ref):
    mn_log = jnp.log10(jnp.float32(MIN_POS_K))
    mx_log = jnp.log10(jnp.float32(MAX_POS_K))
    t = jnp.clip(d_ref[...], MIN_POS_K, MAX_POS_K)
    t = jnp.log10(t)
    t = (t - mn_log) / (mx_log - mn_log)
    t = N_POS_EMB_K * t
    idx = t.astype(jnp.int32) * N_HEADS_K
    br, cols = d_ref.shape
    idx_ref[...] = idx.reshape(br * cols // SUB, SUB)


def _compute_idx(d2):
    rows, cols = d2.shape
    br = 256
    return pl.pallas_call(
        _idx_body,
        grid=(rows // br,),
        in_specs=[pl.BlockSpec((br, cols), lambda i: (i, 0))],
        out_specs=pl.BlockSpec((br * cols // SUB, SUB), lambda i: (i, 0)),
        out_shape=jax.ShapeDtypeStruct((rows * cols // SUB, SUB), jnp.int32),
    )(d2)


def _sc_gather(idx2, table_flat, total):
    b_per_w = total // NW
    nchunks = b_per_w // CHUNK
    nsub = CHUNK // SUB
    mesh = plsc.VectorSubcoreMesh(
        core_axis_name="c", subcore_axis_name="s", num_cores=NC, num_subcores=NS
    )

    @functools.partial(
        pl.kernel,
        out_type=jax.ShapeDtypeStruct((total, N_HEADS_K), jnp.float32),
        mesh=mesh,
        compiler_params=pltpu.CompilerParams(
            needs_layout_passes=False, use_tc_tiling_on_sc=False
        ),
        scratch_types=[
            pltpu.VMEM((TAB_PAD,), jnp.float32),
            pltpu.VMEM((nsub, SUB), jnp.int32),
            pltpu.VMEM((CHUNK, N_HEADS_K), jnp.float32),
        ],
    )
    def run(idx_hbm, table_hbm, out_hbm, tab_v, idx_v, rows_v):
        wid = lax.axis_index("s") * NC + lax.axis_index("c")
        base_w = wid * b_per_w
        pltpu.sync_copy(table_hbm, tab_v)
        lane = lax.iota(jnp.int32, LANES)

        @pl.loop(0, nchunks)
        def _(i):
            base = pl.multiple_of(base_w + i * CHUNK, CHUNK)
            row = pl.multiple_of(base // SUB, nsub)
            pltpu.sync_copy(idx_hbm.at[pl.ds(row, nsub)], idx_v)

            @pl.loop(0, nsub)
            def _(j):
                @pl.loop(0, SUB // LANES)
                def _(q):
                    f = idx_v[j, pl.ds(q * LANES, LANES)]
                    r = lane + (j * SUB + q * LANES)
                    for h in range(N_HEADS_K):
                        vals = plsc.load_gather(tab_v, [f + h])
                        plsc.store_scatter(
                            rows_v, [r, jnp.full((LANES,), h, jnp.int32)], vals
                        )

            pltpu.sync_copy(rows_v, out_hbm.at[pl.ds(base, CHUNK)])

    return run(idx2, table_flat)


def kernel(d_mat, embeddings_table):
    b, rows, cols = d_mat.shape
    total = b * rows * cols
    idx2 = _compute_idx(d_mat.reshape(b * rows, cols))
    tab_flat = jnp.zeros((TAB_PAD,), jnp.float32).at[: (N_POS_EMB_K + 1) * N_HEADS_K].set(
        embeddings_table.reshape(-1)
    )
    out = _sc_gather(idx2, tab_flat, total)
    return out.reshape(b, rows, cols, N_HEADS_K)


# speedup vs baseline: 25.0475x; 3.5119x over previous
import functools

import jax
import jax.numpy as jnp
from jax import lax
from jax.experimental import pallas as pl
from jax.experimental.pallas import tpu as pltpu
from jax.experimental.pallas import tpu_sc as plsc

MIN_POS_K = 0.1
MAX_POS_K = 1000.0
N_POS_EMB_K = 512
N_HEADS_K = 12

NC = 2
NS = 16
NW = NC * NS
LANES = 16

ROWB = 8
COLB = 256
PADV = 520


def _idx_body(d_ref, idx_ref):
    mn_log = jnp.log10(jnp.float32(MIN_POS_K))
    mx_log = jnp.log10(jnp.float32(MAX_POS_K))
    t = jnp.clip(d_ref[...], MIN_POS_K, MAX_POS_K)
    t = jnp.log10(t)
    t = (t - mn_log) / (mx_log - mn_log)
    t = N_POS_EMB_K * t
    idx_ref[...] = t.astype(jnp.int32)


def _compute_idx(d2):
    rows, cols = d2.shape
    br = 256
    return pl.pallas_call(
        _idx_body,
        grid=(rows // br,),
        in_specs=[pl.BlockSpec((br, cols), lambda i: (i, 0))],
        out_specs=pl.BlockSpec((br, cols), lambda i: (i, 0)),
        out_shape=jax.ShapeDtypeStruct((rows, cols), jnp.int32),
    )(d2)


def _sc_gather(idx2, tab_t):
    rows, cols = idx2.shape
    bands = rows // ROWB
    bands_per_w = bands // NW
    ncol = cols // COLB
    mesh = plsc.VectorSubcoreMesh(
        core_axis_name="c", subcore_axis_name="s", num_cores=NC, num_subcores=NS
    )

    @functools.partial(
        pl.kernel,
        out_type=jax.ShapeDtypeStruct((1, N_HEADS_K, rows, cols), jnp.float32),
        mesh=mesh,
        compiler_params=pltpu.CompilerParams(needs_layout_passes=False),
        scratch_types=[
            pltpu.VMEM((N_HEADS_K * PADV,), jnp.float32),
            pltpu.VMEM((ROWB, COLB), jnp.int32),
            pltpu.VMEM((N_HEADS_K, ROWB, COLB), jnp.float32),
        ],
    )
    def run(idx_hbm, table_hbm, out_hbm, tab_v, idx_v, rows_v):
        wid = lax.axis_index("s") * NC + lax.axis_index("c")
        band0 = wid * bands_per_w
        pltpu.sync_copy(table_hbm, tab_v)

        @pl.loop(0, bands_per_w)
        def _(b):
            r0 = pl.multiple_of((band0 + b) * ROWB, ROWB)

            @pl.loop(0, ncol)
            def _(cb):
                c0 = pl.multiple_of(cb * COLB, COLB)
                pltpu.sync_copy(
                    idx_hbm.at[pl.ds(r0, ROWB), pl.ds(c0, COLB)], idx_v
                )

                @pl.loop(0, ROWB)
                def _(ri):
                    @pl.loop(0, COLB // LANES)
                    def _(q):
                        f = idx_v[ri, pl.ds(q * LANES, LANES)]
                        for h in range(N_HEADS_K):
                            vals = plsc.load_gather(tab_v, [f + h * PADV])
                            rows_v[h, ri, pl.ds(q * LANES, LANES)] = vals

                for h in range(N_HEADS_K):
                    pltpu.sync_copy(
                        rows_v.at[h],
                        out_hbm.at[0, h, pl.ds(r0, ROWB), pl.ds(c0, COLB)],
                    )

    return run(idx2, tab_t)


def kernel(d_mat, embeddings_table):
    b, rows, cols = d_mat.shape
    idx2 = _compute_idx(d_mat.reshape(b * rows, cols))
    tab_t = (
        jnp.zeros((N_HEADS_K, PADV), jnp.float32)
        .at[:, : N_POS_EMB_K + 1]
        .set(embeddings_table.T)
        .reshape(-1)
    )
    out = _sc_gather(idx2, tab_t)
    return out.transpose(0, 2, 3, 1)


# speedup vs baseline: 32.1536x; 1.2837x over previous
import functools

import jax
import jax.numpy as jnp
from jax import lax
from jax.experimental import pallas as pl
from jax.experimental.pallas import tpu as pltpu
from jax.experimental.pallas import tpu_sc as plsc

MIN_POS_K = 0.1
MAX_POS_K = 1000.0
N_POS_EMB_K = 512
N_HEADS_K = 12

NC = 2
NS = 16
NW = NC * NS
LANES = 16

ROWB = 8
COLB = 256
PADV = 520


def _idx_body(d_ref, idx_ref):
    mn_log = jnp.log10(jnp.float32(MIN_POS_K))
    mx_log = jnp.log10(jnp.float32(MAX_POS_K))
    t = jnp.clip(d_ref[...], MIN_POS_K, MAX_POS_K)
    t = jnp.log10(t)
    t = (t - mn_log) / (mx_log - mn_log)
    t = N_POS_EMB_K * t
    idx_ref[...] = t.astype(jnp.int32)


def _compute_idx(d2):
    rows, cols = d2.shape
    br = 256
    return pl.pallas_call(
        _idx_body,
        grid=(rows // br,),
        in_specs=[pl.BlockSpec((br, cols), lambda i: (i, 0))],
        out_specs=pl.BlockSpec((br, cols), lambda i: (i, 0)),
        out_shape=jax.ShapeDtypeStruct((rows, cols), jnp.int32),
    )(d2)


def _sc_gather(idx2, tab_t):
    rows, cols = idx2.shape
    bands = rows // ROWB
    bands_per_w = bands // NW
    ncol = cols // COLB
    mesh = plsc.VectorSubcoreMesh(
        core_axis_name="c", subcore_axis_name="s", num_cores=NC, num_subcores=NS
    )

    nchunks = bands_per_w * ncol
    qper = COLB // LANES

    @functools.partial(
        pl.kernel,
        out_type=jax.ShapeDtypeStruct((1, N_HEADS_K, rows, cols), jnp.float32),
        mesh=mesh,
        compiler_params=pltpu.CompilerParams(needs_layout_passes=False),
        scratch_types=[
            pltpu.VMEM((N_HEADS_K * PADV,), jnp.float32),
            pltpu.VMEM((ROWB, COLB), jnp.int32),
            pltpu.VMEM((ROWB, COLB), jnp.int32),
            pltpu.VMEM((N_HEADS_K, ROWB, COLB), jnp.float32),
            pltpu.VMEM((N_HEADS_K, ROWB, COLB), jnp.float32),
            pltpu.SemaphoreType.DMA,
            pltpu.SemaphoreType.DMA,
            pltpu.SemaphoreType.DMA,
            pltpu.SemaphoreType.DMA,
        ],
    )
    def run(idx_hbm, table_hbm, out_hbm, tab_v, idx_v0, idx_v1,
            rows_v0, rows_v1, sin0, sin1, sout0, sout1):
        wid = lax.axis_index("s") * NC + lax.axis_index("c")
        band0 = wid * bands_per_w
        idx_bufs = (idx_v0, idx_v1)
        rows_bufs = (rows_v0, rows_v1)
        sins = (sin0, sin1)
        souts = (sout0, sout1)
        pltpu.sync_copy(table_hbm, tab_v)

        def chunk_slices(k):
            r0 = pl.multiple_of((band0 + k // ncol) * ROWB, ROWB)
            c0 = pl.multiple_of((k % ncol) * COLB, COLB)
            return r0, c0

        def issue_in(k, p):
            r0, c0 = chunk_slices(k)
            pltpu.async_copy(
                idx_hbm.at[pl.ds(r0, ROWB), pl.ds(c0, COLB)], idx_bufs[p], sins[p]
            )

        def wait_in(p):
            pltpu.make_async_copy(
                idx_hbm.at[pl.ds(0, ROWB), pl.ds(0, COLB)], idx_bufs[p], sins[p]
            ).wait()

        def issue_outs(k, p):
            r0, c0 = chunk_slices(k)
            for h in range(N_HEADS_K):
                pltpu.async_copy(
                    rows_bufs[p].at[h],
                    out_hbm.at[0, h, pl.ds(r0, ROWB), pl.ds(c0, COLB)],
                    souts[p],
                )

        def wait_outs(p):
            for h in range(N_HEADS_K):
                pltpu.make_async_copy(
                    rows_bufs[p].at[h],
                    out_hbm.at[0, h, pl.ds(0, ROWB), pl.ds(0, COLB)],
                    souts[p],
                ).wait()

        def compute(p):
            idx_b, rows_b = idx_bufs[p], rows_bufs[p]

            @pl.loop(0, ROWB * qper, unroll=4)
            def _(t):
                ri = t // qper
                q = t % qper
                f = idx_b[ri, pl.ds(q * LANES, LANES)]
                for h in range(N_HEADS_K):
                    vals = plsc.load_gather(tab_v, [f + h * PADV])
                    rows_b[h, ri, pl.ds(q * LANES, LANES)] = vals

        issue_in(0, 0)

        @pl.loop(0, nchunks // 2)
        def _(g):
            for p in range(2):
                k = g * 2 + p

                @pl.when(k + 1 < nchunks)
                def _():
                    issue_in(k + 1, 1 - p)

                wait_in(p)

                @pl.when(k >= 2)
                def _():
                    wait_outs(p)

                compute(p)
                issue_outs(k, p)

        wait_outs(0)
        wait_outs(1)

    return run(idx2, tab_t)


def kernel(d_mat, embeddings_table):
    b, rows, cols = d_mat.shape
    idx2 = _compute_idx(d_mat.reshape(b * rows, cols))
    tab_t = (
        jnp.zeros((N_HEADS_K, PADV), jnp.float32)
        .at[:, : N_POS_EMB_K + 1]
        .set(embeddings_table.T)
        .reshape(-1)
    )
    out = _sc_gather(idx2, tab_t)
    return out.transpose(0, 2, 3, 1)


# speedup vs baseline: 32.3548x; 1.0063x over previous
import functools

import jax
import jax.numpy as jnp
from jax import lax
from jax.experimental import pallas as pl
from jax.experimental.pallas import tpu as pltpu
from jax.experimental.pallas import tpu_sc as plsc

MIN_POS_K = 0.1
MAX_POS_K = 1000.0
N_POS_EMB_K = 512
N_HEADS_K = 12

NC = 2
NS = 16
NW = NC * NS
LANES = 16

ROWB = 8
COLB = 512
PADV = 520


def _idx_body(d_ref, idx_ref):
    mn_log = jnp.log10(jnp.float32(MIN_POS_K))
    mx_log = jnp.log10(jnp.float32(MAX_POS_K))
    t = jnp.clip(d_ref[...], MIN_POS_K, MAX_POS_K)
    t = jnp.log10(t)
    t = (t - mn_log) / (mx_log - mn_log)
    t = N_POS_EMB_K * t
    idx_ref[...] = t.astype(jnp.int32)


def _compute_idx(d2):
    rows, cols = d2.shape
    br = 256
    return pl.pallas_call(
        _idx_body,
        grid=(rows // br,),
        in_specs=[pl.BlockSpec((br, cols), lambda i: (i, 0))],
        out_specs=pl.BlockSpec((br, cols), lambda i: (i, 0)),
        out_shape=jax.ShapeDtypeStruct((rows, cols), jnp.int32),
    )(d2)


def _sc_gather(idx2, tab_t):
    rows, cols = idx2.shape
    bands = rows // ROWB
    bands_per_w = bands // NW
    ncol = cols // COLB
    mesh = plsc.VectorSubcoreMesh(
        core_axis_name="c", subcore_axis_name="s", num_cores=NC, num_subcores=NS
    )

    nchunks = bands_per_w * ncol
    qper = COLB // LANES

    @functools.partial(
        pl.kernel,
        out_type=jax.ShapeDtypeStruct((1, N_HEADS_K, rows, cols), jnp.float32),
        mesh=mesh,
        compiler_params=pltpu.CompilerParams(needs_layout_passes=False),
        scratch_types=[
            pltpu.VMEM((N_HEADS_K * PADV,), jnp.float32),
            pltpu.VMEM((ROWB, COLB), jnp.int32),
            pltpu.VMEM((ROWB, COLB), jnp.int32),
            pltpu.VMEM((N_HEADS_K, ROWB, COLB), jnp.float32),
            pltpu.VMEM((N_HEADS_K, ROWB, COLB), jnp.float32),
            pltpu.SemaphoreType.DMA,
            pltpu.SemaphoreType.DMA,
            pltpu.SemaphoreType.DMA,
            pltpu.SemaphoreType.DMA,
        ],
    )
    def run(idx_hbm, table_hbm, out_hbm, tab_v, idx_v0, idx_v1,
            rows_v0, rows_v1, sin0, sin1, sout0, sout1):
        wid = lax.axis_index("s") * NC + lax.axis_index("c")
        band0 = wid * bands_per_w
        idx_bufs = (idx_v0, idx_v1)
        rows_bufs = (rows_v0, rows_v1)
        sins = (sin0, sin1)
        souts = (sout0, sout1)
        pltpu.sync_copy(table_hbm, tab_v)

        def chunk_slices(k):
            r0 = pl.multiple_of((band0 + k // ncol) * ROWB, ROWB)
            c0 = pl.multiple_of((k % ncol) * COLB, COLB)
            return r0, c0

        def issue_in(k, p):
            r0, c0 = chunk_slices(k)
            pltpu.async_copy(
                idx_hbm.at[pl.ds(r0, ROWB), pl.ds(c0, COLB)], idx_bufs[p], sins[p]
            )

        def wait_in(p):
            pltpu.make_async_copy(
                idx_hbm.at[pl.ds(0, ROWB), pl.ds(0, COLB)], idx_bufs[p], sins[p]
            ).wait()

        def issue_outs(k, p):
            r0, c0 = chunk_slices(k)
            pltpu.async_copy(
                rows_bufs[p],
                out_hbm.at[0, :, pl.ds(r0, ROWB), pl.ds(c0, COLB)],
                souts[p],
            )

        def wait_outs(p):
            pltpu.make_async_copy(
                rows_bufs[p],
                out_hbm.at[0, :, pl.ds(0, ROWB), pl.ds(0, COLB)],
                souts[p],
            ).wait()

        def compute(p):
            idx_b, rows_b = idx_bufs[p], rows_bufs[p]

            @pl.loop(0, ROWB * qper, unroll=4)
            def _(t):
                ri = t // qper
                q = t % qper
                f = idx_b[ri, pl.ds(q * LANES, LANES)]
                for h in range(N_HEADS_K):
                    vals = plsc.load_gather(tab_v, [f + h * PADV])
                    rows_b[h, ri, pl.ds(q * LANES, LANES)] = vals

        issue_in(0, 0)

        @pl.loop(0, nchunks // 2)
        def _(g):
            for p in range(2):
                k = g * 2 + p

                @pl.when(k + 1 < nchunks)
                def _():
                    issue_in(k + 1, 1 - p)

                wait_in(p)

                @pl.when(k >= 2)
                def _():
                    wait_outs(p)

                compute(p)
                issue_outs(k, p)

        wait_outs(0)
        wait_outs(1)

    return run(idx2, tab_t)


def kernel(d_mat, embeddings_table):
    b, rows, cols = d_mat.shape
    idx2 = _compute_idx(d_mat.reshape(b * rows, cols))
    tab_t = (
        jnp.zeros((N_HEADS_K, PADV), jnp.float32)
        .at[:, : N_POS_EMB_K + 1]
        .set(embeddings_table.T)
        .reshape(-1)
    )
    out = _sc_gather(idx2, tab_t)
    return out.transpose(0, 2, 3, 1)


# speedup vs baseline: 129.5241x; 4.0032x over previous
import functools

import jax
import jax.numpy as jnp
from jax import lax
from jax.experimental import pallas as pl
from jax.experimental.pallas import tpu as pltpu
from jax.experimental.pallas import tpu_sc as plsc

MIN_POS_K = 0.1
MAX_POS_K = 1000.0
N_POS_EMB_K = 512
N_HEADS_K = 12

NC = 2
NS = 16
NW = NC * NS
LANES = 16

ROWB = 8
COLB = 512
PADV = 520


def _idx_body(d_ref, idx_ref):
    mn_log = jnp.log10(jnp.float32(MIN_POS_K))
    mx_log = jnp.log10(jnp.float32(MAX_POS_K))
    t = jnp.clip(d_ref[...], MIN_POS_K, MAX_POS_K)
    t = jnp.log10(t)
    t = (t - mn_log) / (mx_log - mn_log)
    t = N_POS_EMB_K * t
    idx_ref[...] = t.astype(jnp.int32)


def _compute_idx(d2):
    rows, cols = d2.shape
    br = 256
    return pl.pallas_call(
        _idx_body,
        grid=(rows // br,),
        in_specs=[pl.BlockSpec((br, cols), lambda i: (i, 0))],
        out_specs=pl.BlockSpec((br, cols), lambda i: (i, 0)),
        out_shape=jax.ShapeDtypeStruct((rows, cols), jnp.int32),
    )(d2)


def _sc_gather(idx2, tab_t):
    rows, cols = idx2.shape
    bands = rows // ROWB
    bands_per_w = bands // NW
    ncol = cols // COLB
    mesh = plsc.VectorSubcoreMesh(
        core_axis_name="c", subcore_axis_name="s", num_cores=NC, num_subcores=NS
    )

    nchunks = bands_per_w * ncol
    qper = COLB // LANES

    @functools.partial(
        pl.kernel,
        out_type=jax.ShapeDtypeStruct((1, N_HEADS_K, rows, cols), jnp.float32),
        mesh=mesh,
        compiler_params=pltpu.CompilerParams(needs_layout_passes=False),
        scratch_types=[
            pltpu.VMEM((N_HEADS_K * PADV,), jnp.float32),
            pltpu.VMEM((ROWB, COLB), jnp.int32),
            pltpu.VMEM((ROWB, COLB), jnp.int32),
            pltpu.VMEM((N_HEADS_K, ROWB, COLB), jnp.float32),
            pltpu.VMEM((N_HEADS_K, ROWB, COLB), jnp.float32),
            pltpu.SemaphoreType.DMA,
            pltpu.SemaphoreType.DMA,
            pltpu.SemaphoreType.DMA,
            pltpu.SemaphoreType.DMA,
        ],
    )
    def run(idx_hbm, table_hbm, out_hbm, tab_v, idx_v0, idx_v1,
            rows_v0, rows_v1, sin0, sin1, sout0, sout1):
        wid = lax.axis_index("s") * NC + lax.axis_index("c")
        band0 = wid * bands_per_w
        idx_bufs = (idx_v0, idx_v1)
        rows_bufs = (rows_v0, rows_v1)
        sins = (sin0, sin1)
        souts = (sout0, sout1)
        pltpu.sync_copy(table_hbm, tab_v)

        def chunk_slices(k):
            r0 = pl.multiple_of((band0 + k // ncol) * ROWB, ROWB)
            c0 = pl.multiple_of((k % ncol) * COLB, COLB)
            return r0, c0

        def issue_in(k, p):
            r0, c0 = chunk_slices(k)
            pltpu.async_copy(
                idx_hbm.at[pl.ds(r0, ROWB), pl.ds(c0, COLB)], idx_bufs[p], sins[p]
            )

        def wait_in(p):
            pltpu.make_async_copy(
                idx_hbm.at[pl.ds(0, ROWB), pl.ds(0, COLB)], idx_bufs[p], sins[p]
            ).wait()

        def issue_outs(k, p):
            r0, c0 = chunk_slices(k)
            pltpu.async_copy(
                rows_bufs[p],
                out_hbm.at[0, :, pl.ds(r0, ROWB), pl.ds(c0, COLB)],
                souts[p],
            )

        def wait_outs(p):
            pltpu.make_async_copy(
                rows_bufs[p],
                out_hbm.at[0, :, pl.ds(0, ROWB), pl.ds(0, COLB)],
                souts[p],
            ).wait()

        def compute(p):
            idx_b, rows_b = idx_bufs[p], rows_bufs[p]

            @plsc.parallel_loop(0, ROWB * qper, unroll=4)
            def _(t):
                ri = t // qper
                q = t % qper
                f = idx_b[ri, pl.ds(q * LANES, LANES)]
                for h in range(N_HEADS_K):
                    vals = plsc.load_gather(tab_v, [f + h * PADV])
                    rows_b[h, ri, pl.ds(q * LANES, LANES)] = vals

        issue_in(0, 0)

        @pl.loop(0, nchunks // 2)
        def _(g):
            for p in range(2):
                k = g * 2 + p

                @pl.when(k + 1 < nchunks)
                def _():
                    issue_in(k + 1, 1 - p)

                wait_in(p)

                @pl.when(k >= 2)
                def _():
                    wait_outs(p)

                compute(p)
                issue_outs(k, p)

        wait_outs(0)
        wait_outs(1)

    return run(idx2, tab_t)


def kernel(d_mat, embeddings_table):
    b, rows, cols = d_mat.shape
    idx2 = _compute_idx(d_mat.reshape(b * rows, cols))
    tab_t = (
        jnp.zeros((N_HEADS_K, PADV), jnp.float32)
        .at[:, : N_POS_EMB_K + 1]
        .set(embeddings_table.T)
        .reshape(-1)
    )
    out = _sc_gather(idx2, tab_t)
    return out.transpose(0, 2, 3, 1)
